# baseline (speedup 1.0000x reference)
import functools

import jax
import jax.numpy as jnp
from jax import lax
from jax.experimental import pallas as pl
from jax.experimental.pallas import tpu as pltpu
from jax.experimental.pallas import tpu_sc as plsc

NC = 2
NS = 16
NW = NC * NS
CHUNK = 128
NBUF = 4

_MESH = dict(core_axis_name="c", subcore_axis_name="s", num_cores=NC,
             num_subcores=NS)


def _deg_kernel_body(cpw_deg, rpt, col_hbm, deg_out, colv, ones_v, zer_v,
                     hist):
  cid = lax.axis_index("c")
  sid = lax.axis_index("s")
  wid = sid * NC + cid

  for i in range(CHUNK // 16):
    ones_v[pl.ds(i * 16, 16)] = jnp.ones((16,), jnp.float32)
  for i in range(rpt // 16):
    zer_v[pl.ds(i * 16, 16)] = jnp.zeros((16,), jnp.float32)
  pltpu.sync_copy(zer_v, hist.at[pl.ds(sid * rpt, rpt)])
  pltpu.sync_copy(col_hbm.at[wid], colv)
  plsc.subcore_barrier()

  def body(j, carry):
    pltpu.sync_copy(ones_v, hist.at[colv.at[j]], add=True)
    return carry

  lax.fori_loop(0, cpw_deg, body, 0)
  plsc.subcore_barrier()
  pltpu.sync_copy(hist.at[pl.ds(sid * rpt, rpt)],
                  deg_out.at[cid, pl.ds(sid * rpt, rpt)])


def _agg_kernel_body(cpw, rpt, hd, y0_hbm, y1_hbm, rc_hbm, z_hbm,
                     out0_hbm, out1_hbm, rcv, buf, acc, gsems, ssems):
  cid = lax.axis_index("c")
  sid = lax.axis_index("s")

  pltpu.sync_copy(z_hbm, acc.at[pl.ds(sid * rpt, rpt)])
  pltpu.sync_copy(rc_hbm.at[sid], rcv)
  plsc.subcore_barrier()

  def prime(j, carry):
    @pl.when(cid == 0)
    def _():
      pltpu.async_copy(y0_hbm.at[rcv.at[0, j]], buf.at[j], gsems.at[j])

    @pl.when(cid == 1)
    def _():
      pltpu.async_copy(y1_hbm.at[rcv.at[0, j]], buf.at[j], gsems.at[j])

    return carry

  lax.fori_loop(0, 2, prime, 0)

  def body(j, carry):
    par = lax.rem(j, NBUF)
    par2 = lax.rem(j + 2, NBUF)
    pltpu.make_async_copy(y0_hbm.at[pl.ds(0, CHUNK)], buf.at[par],
                          gsems.at[par]).wait()
    pltpu.async_copy(buf.at[par], acc.at[rcv.at[1, j]], ssems.at[par],
                     add=True)

    @pl.when(j >= 2)
    def _():
      pltpu.make_async_copy(y0_hbm.at[pl.ds(0, CHUNK)], buf.at[par2],
                            ssems.at[par2]).wait()

    @pl.when(jnp.logical_and(j + 2 < cpw, cid == 0))
    def _():
      pltpu.async_copy(y0_hbm.at[rcv.at[0, j + 2]], buf.at[par2],
                       gsems.at[par2])

    @pl.when(jnp.logical_and(j + 2 < cpw, cid == 1))
    def _():
      pltpu.async_copy(y1_hbm.at[rcv.at[0, j + 2]], buf.at[par2],
                       gsems.at[par2])

    return carry

  lax.fori_loop(0, cpw, body, 0)

  def drain(i, carry):
    par = lax.rem(cpw - 2 + i, NBUF)
    pltpu.make_async_copy(y0_hbm.at[pl.ds(0, CHUNK)], buf.at[par],
                          ssems.at[par]).wait()
    return carry

  lax.fori_loop(0, 2, drain, 0)
  plsc.subcore_barrier()

  @pl.when(cid == 0)
  def _():
    pltpu.sync_copy(acc.at[pl.ds(sid * rpt, rpt)],
                    out0_hbm.at[pl.ds(sid * rpt, rpt)])

  @pl.when(cid == 1)
  def _():
    pltpu.sync_copy(acc.at[pl.ds(sid * rpt, rpt)],
                    out1_hbm.at[pl.ds(sid * rpt, rpt)])


def _matmul_body(xb, wb, xwb):
  xwb[...] = jnp.dot(xb[...], wb[...], preferred_element_type=jnp.float32)


def _scale_body(xwb, degb, yb0, yb1):
  deg = degb[0, :] + degb[1, :] + 1.0
  dis = lax.rsqrt(deg)
  res = xwb[...] * dis[:, None]
  hd = res.shape[1] // 2
  yb0[...] = res[:, :hd]
  yb1[...] = res[:, hd:]


def _finish_body(a0b, a1b, y0b, y1b, degb, bb, ab, zb):
  deg = degb[0, :] + degb[1, :] + 1.0
  dis = lax.rsqrt(deg)
  acc = jnp.concatenate([a0b[...] + y0b[...], a1b[...] + y1b[...]], axis=1)
  s = acc * dis[:, None] + bb[...]
  zb[...] = jnp.where(s >= 0, s, ab[...] * s)


def kernel(x, edge_index, W, b, alpha):
  n = x.shape[0]
  e = edge_index.shape[1]
  d = x.shape[1]
  hd = d // 2

  nacc = 10240
  rpt = nacc // NS

  cpw = -(-e // (NS * CHUNK))
  cpw += -cpw % 2
  epad = NS * cpw * CHUNK - e
  row = jnp.concatenate(
      [edge_index[0], jnp.zeros((epad,), jnp.int32)]).reshape(NS, 1, cpw,
                                                             CHUNK)
  pad_cols = n + jnp.arange(epad, dtype=jnp.int32) % (nacc - n)
  col = jnp.concatenate(
      [edge_index[1], pad_cols]).reshape(NS, 1, cpw, CHUNK)
  rc = jnp.concatenate([row, col], axis=1)

  cpw_deg = -(-e // (NW * CHUNK))
  epad_deg = NW * cpw_deg * CHUNK - e
  pad_cols_deg = n + jnp.arange(epad_deg, dtype=jnp.int32) % (nacc - n)
  col_deg = jnp.concatenate(
      [edge_index[1], pad_cols_deg]).reshape(NW, cpw_deg, CHUNK)

  zeros_rows = jnp.zeros((rpt, hd), jnp.float32)

  mesh = plsc.VectorSubcoreMesh(**_MESH)

  deg_fn = pl.kernel(
      functools.partial(_deg_kernel_body, cpw_deg, rpt),
      out_type=jax.ShapeDtypeStruct((NC, nacc), jnp.float32),
      mesh=mesh,
      scratch_types=[
          pltpu.VMEM((cpw_deg, CHUNK), jnp.int32),
          pltpu.VMEM((CHUNK,), jnp.float32),
          pltpu.VMEM((rpt,), jnp.float32),
          pltpu.VMEM_SHARED((nacc,), jnp.float32),
      ])
  deg = deg_fn(col_deg)

  xw = pl.pallas_call(
      _matmul_body,
      grid=(nacc // 1024,),
      in_specs=[
          pl.BlockSpec((1024, d), lambda i: (i, 0)),
          pl.BlockSpec((d, d), lambda i: (0, 0)),
      ],
      out_specs=pl.BlockSpec((1024, d), lambda i: (i, 0)),
      out_shape=jax.ShapeDtypeStruct((nacc, d), jnp.float32),
  )(x, W)

  y0, y1 = pl.pallas_call(
      _scale_body,
      grid=(nacc // 1024,),
      in_specs=[
          pl.BlockSpec((1024, d), lambda i: (i, 0)),
          pl.BlockSpec((NC, 1024), lambda i: (0, i)),
      ],
      out_specs=[
          pl.BlockSpec((1024, hd), lambda i: (i, 0)),
          pl.BlockSpec((1024, hd), lambda i: (i, 0)),
      ],
      out_shape=[
          jax.ShapeDtypeStruct((nacc, hd), jnp.float32),
          jax.ShapeDtypeStruct((nacc, hd), jnp.float32),
      ],
  )(xw, deg)

  agg_fn = pl.kernel(
      functools.partial(_agg_kernel_body, cpw, rpt, hd),
      out_type=[
          jax.ShapeDtypeStruct((nacc, hd), jnp.float32),
          jax.ShapeDtypeStruct((nacc, hd), jnp.float32),
      ],
      mesh=mesh,
      scratch_types=[
          pltpu.VMEM((2, cpw, CHUNK), jnp.int32),
          pltpu.VMEM((NBUF, CHUNK, hd), jnp.float32),
          pltpu.VMEM_SHARED((nacc, hd), jnp.float32),
          pltpu.SemaphoreType.DMA((NBUF,)),
          pltpu.SemaphoreType.DMA((NBUF,)),
      ],
      compiler_params=pltpu.CompilerParams(use_tc_tiling_on_sc=False))
  acc0, acc1 = agg_fn(y0, y1, rc, zeros_rows)

  z = pl.pallas_call(
      _finish_body,
      grid=(nacc // 1024,),
      in_specs=[
          pl.BlockSpec((1024, hd), lambda i: (i, 0)),
          pl.BlockSpec((1024, hd), lambda i: (i, 0)),
          pl.BlockSpec((1024, hd), lambda i: (i, 0)),
          pl.BlockSpec((1024, hd), lambda i: (i, 0)),
          pl.BlockSpec((NC, 1024), lambda i: (0, i)),
          pl.BlockSpec((1, d), lambda i: (0, 0)),
          pl.BlockSpec((1, d), lambda i: (0, 0)),
      ],
      out_specs=pl.BlockSpec((1024, d), lambda i: (i, 0)),
      out_shape=jax.ShapeDtypeStruct((n, d), jnp.float32),
  )(acc0, acc1, y0, y1, deg, b.reshape(1, d), alpha.reshape(1, d))

  return z


# speedup vs baseline: 1.0541x; 1.0541x over previous
import functools

import jax
import jax.numpy as jnp
from jax import lax
from jax.experimental import pallas as pl
from jax.experimental.pallas import tpu as pltpu
from jax.experimental.pallas import tpu_sc as plsc

NC = 2
NS = 16
NW = NC * NS
CHUNK = 128
NBUF = 6
PFD = 3

_MESH = dict(core_axis_name="c", subcore_axis_name="s", num_cores=NC,
             num_subcores=NS)


def _deg_kernel_body(cpw_deg, rpt, col_hbm, deg_out, colv, ones_v, zer_v,
                     hist):
  cid = lax.axis_index("c")
  sid = lax.axis_index("s")
  wid = sid * NC + cid

  for i in range(CHUNK // 16):
    ones_v[pl.ds(i * 16, 16)] = jnp.ones((16,), jnp.float32)
  for i in range(rpt // 16):
    zer_v[pl.ds(i * 16, 16)] = jnp.zeros((16,), jnp.float32)
  pltpu.sync_copy(zer_v, hist.at[pl.ds(sid * rpt, rpt)])
  pltpu.sync_copy(col_hbm.at[wid], colv)
  plsc.subcore_barrier()

  def body(j, carry):
    pltpu.sync_copy(ones_v, hist.at[colv.at[j]], add=True)
    return carry

  lax.fori_loop(0, cpw_deg, body, 0)
  plsc.subcore_barrier()
  pltpu.sync_copy(hist.at[pl.ds(sid * rpt, rpt)],
                  deg_out.at[cid, pl.ds(sid * rpt, rpt)])


def _agg_kernel_body(cpw, rpt, hd, y0_hbm, y1_hbm, rc_hbm, z_hbm,
                     out0_hbm, out1_hbm, rcv, buf, acc, gsems, ssems):
  cid = lax.axis_index("c")
  sid = lax.axis_index("s")

  pltpu.sync_copy(z_hbm, acc.at[pl.ds(sid * rpt, rpt)])
  pltpu.sync_copy(rc_hbm.at[sid], rcv)
  plsc.subcore_barrier()

  def prime(j, carry):
    @pl.when(cid == 0)
    def _():
      pltpu.async_copy(y0_hbm.at[rcv.at[0, j]], buf.at[j], gsems.at[j])

    @pl.when(cid == 1)
    def _():
      pltpu.async_copy(y1_hbm.at[rcv.at[0, j]], buf.at[j], gsems.at[j])

    return carry

  lax.fori_loop(0, PFD, prime, 0)

  def body(j, carry):
    par = lax.rem(j, NBUF)
    par2 = lax.rem(j + PFD, NBUF)
    pltpu.make_async_copy(y0_hbm.at[pl.ds(0, CHUNK)], buf.at[par],
                          gsems.at[par]).wait()
    pltpu.async_copy(buf.at[par], acc.at[rcv.at[1, j]], ssems.at[par],
                     add=True)

    @pl.when(j >= PFD)
    def _():
      pltpu.make_async_copy(y0_hbm.at[pl.ds(0, CHUNK)], buf.at[par2],
                            ssems.at[par2]).wait()

    @pl.when(jnp.logical_and(j + PFD < cpw, cid == 0))
    def _():
      pltpu.async_copy(y0_hbm.at[rcv.at[0, j + PFD]], buf.at[par2],
                       gsems.at[par2])

    @pl.when(jnp.logical_and(j + PFD < cpw, cid == 1))
    def _():
      pltpu.async_copy(y1_hbm.at[rcv.at[0, j + PFD]], buf.at[par2],
                       gsems.at[par2])

    return carry

  lax.fori_loop(0, cpw, body, 0)

  def drain(i, carry):
    par = lax.rem(cpw - PFD + i, NBUF)
    pltpu.make_async_copy(y0_hbm.at[pl.ds(0, CHUNK)], buf.at[par],
                          ssems.at[par]).wait()
    return carry

  lax.fori_loop(0, PFD, drain, 0)
  plsc.subcore_barrier()

  @pl.when(cid == 0)
  def _():
    pltpu.sync_copy(acc.at[pl.ds(sid * rpt, rpt)],
                    out0_hbm.at[pl.ds(sid * rpt, rpt)])

  @pl.when(cid == 1)
  def _():
    pltpu.sync_copy(acc.at[pl.ds(sid * rpt, rpt)],
                    out1_hbm.at[pl.ds(sid * rpt, rpt)])


def _matmul_body(xb, degb, wb, yb0, yb1):
  deg = degb[0, :] + degb[1, :] + 1.0
  dis = lax.rsqrt(deg)
  res = jnp.dot(xb[...], wb[...],
                preferred_element_type=jnp.float32) * dis[:, None]
  hd = res.shape[1] // 2
  yb0[...] = res[:, :hd]
  yb1[...] = res[:, hd:]


def _finish_body(a0b, a1b, y0b, y1b, degb, bb, ab, zb):
  deg = degb[0, :] + degb[1, :] + 1.0
  dis = lax.rsqrt(deg)
  acc = jnp.concatenate([a0b[...] + y0b[...], a1b[...] + y1b[...]], axis=1)
  s = acc * dis[:, None] + bb[...]
  zb[...] = jnp.where(s >= 0, s, ab[...] * s)


def kernel(x, edge_index, W, b, alpha):
  n = x.shape[0]
  e = edge_index.shape[1]
  d = x.shape[1]
  hd = d // 2

  nacc = 10240
  rpt = nacc // NS

  cpw = -(-e // (NS * CHUNK))
  cpw += -cpw % 2
  epad = NS * cpw * CHUNK - e
  row = jnp.concatenate(
      [edge_index[0], jnp.zeros((epad,), jnp.int32)]).reshape(NS, 1, cpw,
                                                             CHUNK)
  pad_cols = n + jnp.arange(epad, dtype=jnp.int32) % (nacc - n)
  col = jnp.concatenate(
      [edge_index[1], pad_cols]).reshape(NS, 1, cpw, CHUNK)
  rc = jnp.concatenate([row, col], axis=1)

  cpw_deg = -(-e // (NW * CHUNK))
  epad_deg = NW * cpw_deg * CHUNK - e
  pad_cols_deg = n + jnp.arange(epad_deg, dtype=jnp.int32) % (nacc - n)
  col_deg = jnp.concatenate(
      [edge_index[1], pad_cols_deg]).reshape(NW, cpw_deg, CHUNK)

  zeros_rows = jnp.zeros((rpt, hd), jnp.float32)

  mesh = plsc.VectorSubcoreMesh(**_MESH)

  deg_fn = pl.kernel(
      functools.partial(_deg_kernel_body, cpw_deg, rpt),
      out_type=jax.ShapeDtypeStruct((NC, nacc), jnp.float32),
      mesh=mesh,
      scratch_types=[
          pltpu.VMEM((cpw_deg, CHUNK), jnp.int32),
          pltpu.VMEM((CHUNK,), jnp.float32),
          pltpu.VMEM((rpt,), jnp.float32),
          pltpu.VMEM_SHARED((nacc,), jnp.float32),
      ])
  deg = deg_fn(col_deg)

  y0, y1 = pl.pallas_call(
      _matmul_body,
      grid=(nacc // 1024,),
      in_specs=[
          pl.BlockSpec((1024, d), lambda i: (i, 0)),
          pl.BlockSpec((NC, 1024), lambda i: (0, i)),
          pl.BlockSpec((d, d), lambda i: (0, 0)),
      ],
      out_specs=[
          pl.BlockSpec((1024, hd), lambda i: (i, 0)),
          pl.BlockSpec((1024, hd), lambda i: (i, 0)),
      ],
      out_shape=[
          jax.ShapeDtypeStruct((nacc, hd), jnp.float32),
          jax.ShapeDtypeStruct((nacc, hd), jnp.float32),
      ],
  )(x, deg, W)

  agg_fn = pl.kernel(
      functools.partial(_agg_kernel_body, cpw, rpt, hd),
      out_type=[
          jax.ShapeDtypeStruct((nacc, hd), jnp.float32),
          jax.ShapeDtypeStruct((nacc, hd), jnp.float32),
      ],
      mesh=mesh,
      scratch_types=[
          pltpu.VMEM((2, cpw, CHUNK), jnp.int32),
          pltpu.VMEM((NBUF, CHUNK, hd), jnp.float32),
          pltpu.VMEM_SHARED((nacc, hd), jnp.float32),
          pltpu.SemaphoreType.DMA((NBUF,)),
          pltpu.SemaphoreType.DMA((NBUF,)),
      ],
      compiler_params=pltpu.CompilerParams(use_tc_tiling_on_sc=False))
  acc0, acc1 = agg_fn(y0, y1, rc, zeros_rows)

  z = pl.pallas_call(
      _finish_body,
      grid=(nacc // 1024,),
      in_specs=[
          pl.BlockSpec((1024, hd), lambda i: (i, 0)),
          pl.BlockSpec((1024, hd), lambda i: (i, 0)),
          pl.BlockSpec((1024, hd), lambda i: (i, 0)),
          pl.BlockSpec((1024, hd), lambda i: (i, 0)),
          pl.BlockSpec((NC, 1024), lambda i: (0, i)),
          pl.BlockSpec((1, d), lambda i: (0, 0)),
          pl.BlockSpec((1, d), lambda i: (0, 0)),
      ],
      out_specs=pl.BlockSpec((1024, d), lambda i: (i, 0)),
      out_shape=jax.ShapeDtypeStruct((n, d), jnp.float32),
  )(acc0, acc1, y0, y1, deg, b.reshape(1, d), alpha.reshape(1, d))

  return z
